# baseline (speedup 1.0000x reference)
import math

import jax
import jax.numpy as jnp
import numpy as np
from jax.experimental import pallas as pl
from jax.experimental.pallas import tpu as pltpu
from jax.experimental.pallas import tpu_sc as plsc
import functools

B, S, D = 1, 2048, 768
H = 12
DH = D // H
DFF = 2048
E = 8
PHI = (1.0 + math.sqrt(5.0)) / 2.0

_r = np.array(
    [[1.0, -1.0, 0.0, 0.0],
     [0.0, 1.0, -1.0, 0.0],
     [0.0, 0.0, 1.0, 0.0],
     [-0.5, 0.5, 0.5, PHI / 2.0]], dtype=np.float32)
_r = _r / np.linalg.norm(_r, axis=1, keepdims=True)
_ROOTS = jnp.asarray(_r)

_pairs = []
for _i in range(16):
    _e1 = _i % E
    _e2 = (_i // 2 + 1) % E
    if _e1 == _e2:
        _e2 = (_e2 + 1) % E
    _pairs.append([_e1, _e2])
_C2E = jnp.asarray(np.array(_pairs, dtype=np.int32))


def _layernorm(x, w, b):
    m = jnp.mean(x, axis=-1, keepdims=True)
    v = jnp.mean((x - m) ** 2, axis=-1, keepdims=True)
    return (x - m) / jnp.sqrt(v + 1e-5) * w + b


def _attention(x, in_proj_w, in_proj_b, out_w, out_b):
    qkv = x @ in_proj_w.T + in_proj_b
    q, k, v = jnp.split(qkv, 3, axis=-1)

    def sh(t):
        return t.reshape(B, S, H, DH).transpose(0, 2, 1, 3)

    q, k, v = sh(q), sh(k), sh(v)
    scores = (q @ k.transpose(0, 1, 3, 2)) / math.sqrt(DH)
    a = jax.nn.softmax(scores, axis=-1)
    o = a @ v
    o = o.transpose(0, 2, 1, 3).reshape(B, S, D)
    return o @ out_w.T + out_b


def _alpha_kernel(wu_ref, wd_ref, au_ref, ad_ref):
    au_ref[...] = jnp.mean(jnp.abs(wu_ref[0])).reshape(1, 1, 1)
    ad_ref[...] = jnp.mean(jnp.abs(wd_ref[0])).reshape(1, 1, 1)


def _erf(x):
    a1, a2, a3, a4, a5 = (0.254829592, -0.284496736, 1.421413741,
                          -1.453152027, 1.061405429)
    p = 0.3275911
    ax = jnp.abs(x)
    t = 1.0 / (1.0 + p * ax)
    poly = t * (a1 + t * (a2 + t * (a3 + t * (a4 + t * a5))))
    y = 1.0 - poly * jnp.exp(-ax * ax)
    return jnp.sign(x) * y


def _gelu(x):
    return 0.5 * x * (1.0 + jax.lax.erf(x * (1.0 / math.sqrt(2.0))))


_NW = 32
_RPW = S // _NW


@functools.partial(
    pl.kernel,
    mesh=plsc.VectorSubcoreMesh(core_axis_name="c", subcore_axis_name="s"),
    out_type=[
        jax.ShapeDtypeStruct((S, D), jnp.float32),
        jax.ShapeDtypeStruct((S, D), jnp.float32),
    ],
    scratch_types=[
        pltpu.VMEM((_RPW,), jnp.int32),
        pltpu.VMEM((_RPW, D), jnp.float32),
        pltpu.SemaphoreType.DMA,
    ],
)
def _sc_combine(us_hbm, p0_hbm, p1_hbm, g0_hbm, g1_hbm, idx_v, rows_v, sem):
    wid = jax.lax.axis_index("s") * 2 + jax.lax.axis_index("c")
    base = wid * _RPW
    pltpu.sync_copy(p0_hbm.at[pl.ds(base, _RPW)], idx_v)
    pltpu.async_copy(us_hbm.at[idx_v], rows_v, sem).wait()
    pltpu.sync_copy(rows_v, g0_hbm.at[pl.ds(base, _RPW)])
    pltpu.sync_copy(p1_hbm.at[pl.ds(base, _RPW)], idx_v)
    pltpu.async_copy(us_hbm.at[idx_v], rows_v, sem).wait()
    pltpu.sync_copy(rows_v, g1_hbm.at[pl.ds(base, _RPW)])


TM = 256
P = 2 * S + E * TM
G = P // TM

_RPD = P // _NW
_RPD2 = _RPD // 2


@functools.partial(
    pl.kernel,
    mesh=plsc.VectorSubcoreMesh(core_axis_name="c", subcore_axis_name="s"),
    out_type=jax.ShapeDtypeStruct((P, D), jnp.float32),
    scratch_types=[
        pltpu.VMEM((_RPD2,), jnp.int32),
        pltpu.VMEM((_RPD2, D), jnp.float32),
        pltpu.SemaphoreType.DMA,
    ],
)
def _sc_dispatch(xn_hbm, tokp_hbm, xg_hbm, idx_v, rows_v, sem):
    wid = jax.lax.axis_index("s") * 2 + jax.lax.axis_index("c")
    for c in range(2):
        base = wid * _RPD + c * _RPD2
        pltpu.sync_copy(tokp_hbm.at[pl.ds(base, _RPD2)], idx_v)
        pltpu.async_copy(xn_hbm.at[idx_v], rows_v, sem).wait()
        pltpu.sync_copy(rows_v, xg_hbm.at[pl.ds(base, _RPD2)])


def _group_kernel(te_ref, xg_ref, wu_ref, wd_ref, au_ref, ad_ref,
                  w_ref, us_ref, wu_s, wd_s):
    i = pl.program_id(0)
    te_prev = te_ref[jnp.maximum(i - 1, 0)]
    te_cur = te_ref[i]

    @pl.when((i == 0) | (te_cur != te_prev))
    def _tern():
        au = au_ref[0, 0, 0]
        ad = ad_ref[0, 0, 0]

        def body(j, carry):
            ru = j * (DFF // 8)
            wu = wu_ref[0, pl.ds(ru, DFF // 8), :]
            wu_s[pl.ds(ru, DFF // 8), :] = jnp.where(
                jnp.abs(wu) > 0.5 * au, jnp.sign(wu), 0.0).astype(jnp.bfloat16)
            rd = j * (D // 8)
            wd = wd_ref[0, pl.ds(rd, D // 8), :]
            wd_s[pl.ds(rd, D // 8), :] = jnp.where(
                jnp.abs(wd) > 0.5 * ad, jnp.sign(wd), 0.0).astype(jnp.bfloat16)
            return carry

        jax.lax.fori_loop(0, 8, body, 0)

    xn = xg_ref[...]
    rms = jnp.sqrt(jnp.mean(xn * xn, axis=-1, keepdims=True))
    xs = (xn / (rms + 1e-8)).astype(jnp.bfloat16)
    u = jax.lax.dot_general(xs, wu_s[...], (((1,), (1,)), ((), ())),
                            preferred_element_type=jnp.float32)
    u = _gelu(u)
    urms = jnp.sqrt(jnp.mean(u * u, axis=-1, keepdims=True))
    u = (u / (urms + 1e-8)).astype(jnp.bfloat16)
    u = jax.lax.dot_general(u, wd_s[...], (((1,), (1,)), ((), ())),
                            preferred_element_type=jnp.float32)
    us_ref[...] = u * w_ref[...]


def kernel(x, in_proj_w, in_proj_b, out_w, out_b, ln1_w, ln1_b,
           ln2_w, ln2_b, W_up, W_down, router_w):
    residual = x
    h = _layernorm(x, ln1_w, ln1_b)
    h = _attention(h, in_proj_w, in_proj_b, out_w, out_b)
    x2 = h + residual
    x_norm = _layernorm(x2, ln2_w, ln2_b)
    h4 = x_norm @ router_w.T
    h4 = h4 / jnp.maximum(jnp.linalg.norm(h4, axis=-1, keepdims=True), 1e-12)
    dots = h4 @ _ROOTS.T
    bits = (dots >= 0).astype(jnp.int32)
    chamber = (bits[..., 0] + 2 * bits[..., 1]
               + 4 * bits[..., 2] + 8 * bits[..., 3])
    expert_indices = _C2E[chamber]
    confidence = jnp.min(jnp.abs(dots), axis=-1)
    w1 = 0.5 + 0.3 * jax.nn.sigmoid(confidence)
    e0 = expert_indices[..., 0].reshape(S, 1)
    e1 = expert_indices[..., 1].reshape(S, 1)
    w1 = w1.reshape(S, 1)

    alpha_up, alpha_dn = pl.pallas_call(
        _alpha_kernel,
        grid=(E,),
        in_specs=[
            pl.BlockSpec((1, DFF, D), lambda e: (e, 0, 0)),
            pl.BlockSpec((1, D, DFF), lambda e: (e, 0, 0)),
        ],
        out_specs=[
            pl.BlockSpec((1, 1, 1), lambda e: (e, 0, 0)),
            pl.BlockSpec((1, 1, 1), lambda e: (e, 0, 0)),
        ],
        out_shape=[
            jax.ShapeDtypeStruct((E, 1, 1), jnp.float32),
            jax.ShapeDtypeStruct((E, 1, 1), jnp.float32),
        ],
    )(W_up, W_down)

    e0f = e0.reshape(S)
    e1f = e1.reshape(S)
    w1f = w1.reshape(S)
    a = jnp.concatenate([e0f, e1f])
    onehot = (a[:, None] == jnp.arange(E, dtype=jnp.int32)[None, :]
              ).astype(jnp.int32)
    counts = jnp.sum(onehot, axis=0)
    rank = jnp.sum((jnp.cumsum(onehot, axis=0) - onehot) * onehot, axis=1)
    cap = ((counts + TM - 1) // TM) * TM
    padded_start = jnp.concatenate(
        [jnp.zeros((1,), jnp.int32), jnp.cumsum(cap)[:-1].astype(jnp.int32)])
    padded_pos = (jnp.sum(onehot * padded_start[None, :], axis=1) + rank
                  ).astype(jnp.int32)
    w_assign = jnp.concatenate([w1f, 1.0 - w1f])
    w_padded = jnp.zeros((P,), jnp.float32).at[padded_pos].set(
        w_assign).reshape(P, 1)
    p0 = padded_pos[:S]
    p1 = padded_pos[S:]
    tile_start = padded_start // TM
    tile_expert = (jnp.sum(
        jnp.arange(G, dtype=jnp.int32)[:, None] >= tile_start[None, :],
        axis=1) - 1).astype(jnp.int32)

    xn_flat = x_norm.reshape(S, D)
    toks = jnp.arange(S, dtype=jnp.int32)
    tok_padded = jnp.zeros((P,), jnp.int32).at[padded_pos].set(
        jnp.concatenate([toks, toks]))
    xg = _sc_dispatch(xn_flat, tok_padded)

    us = pl.pallas_call(
        _group_kernel,
        grid_spec=pltpu.PrefetchScalarGridSpec(
            num_scalar_prefetch=1,
            grid=(G,),
            in_specs=[
                pl.BlockSpec((TM, D), lambda i, te: (i, 0)),
                pl.BlockSpec((1, DFF, D), lambda i, te: (te[i], 0, 0)),
                pl.BlockSpec((1, D, DFF), lambda i, te: (te[i], 0, 0)),
                pl.BlockSpec((1, 1, 1), lambda i, te: (te[i], 0, 0)),
                pl.BlockSpec((1, 1, 1), lambda i, te: (te[i], 0, 0)),
                pl.BlockSpec((TM, 1), lambda i, te: (i, 0)),
            ],
            out_specs=pl.BlockSpec((TM, D), lambda i, te: (i, 0)),
            scratch_shapes=[
                pltpu.VMEM((DFF, D), jnp.bfloat16),
                pltpu.VMEM((D, DFF), jnp.bfloat16),
            ],
        ),
        out_shape=jax.ShapeDtypeStruct((P, D), jnp.float32),
    )(tile_expert, xg, W_up, W_down, alpha_up, alpha_dn, w_padded)

    g0, g1 = _sc_combine(us, p0, p1)
    out = x2.reshape(S, D) + g0 + g1
    return out.reshape(B, S, D)


# speedup vs baseline: 1.2237x; 1.2237x over previous
import math

import jax
import jax.numpy as jnp
import numpy as np
from jax.experimental import pallas as pl
from jax.experimental.pallas import tpu as pltpu
from jax.experimental.pallas import tpu_sc as plsc
import functools

B, S, D = 1, 2048, 768
H = 12
DH = D // H
DFF = 2048
E = 8
PHI = (1.0 + math.sqrt(5.0)) / 2.0

_r = np.array(
    [[1.0, -1.0, 0.0, 0.0],
     [0.0, 1.0, -1.0, 0.0],
     [0.0, 0.0, 1.0, 0.0],
     [-0.5, 0.5, 0.5, PHI / 2.0]], dtype=np.float32)
_r = _r / np.linalg.norm(_r, axis=1, keepdims=True)
_ROOTS = jnp.asarray(_r)

_pairs = []
for _i in range(16):
    _e1 = _i % E
    _e2 = (_i // 2 + 1) % E
    if _e1 == _e2:
        _e2 = (_e2 + 1) % E
    _pairs.append([_e1, _e2])
_C2E = jnp.asarray(np.array(_pairs, dtype=np.int32))


def _layernorm(x, w, b):
    m = jnp.mean(x, axis=-1, keepdims=True)
    v = jnp.mean((x - m) ** 2, axis=-1, keepdims=True)
    return (x - m) / jnp.sqrt(v + 1e-5) * w + b


def _attention(x, in_proj_w, in_proj_b, out_w, out_b):
    qkv = x @ in_proj_w.T + in_proj_b
    q, k, v = jnp.split(qkv, 3, axis=-1)

    def sh(t):
        return t.reshape(B, S, H, DH).transpose(0, 2, 1, 3)

    q, k, v = sh(q), sh(k), sh(v)
    scores = (q @ k.transpose(0, 1, 3, 2)) / math.sqrt(DH)
    a = jax.nn.softmax(scores, axis=-1)
    o = a @ v
    o = o.transpose(0, 2, 1, 3).reshape(B, S, D)
    return o @ out_w.T + out_b


def _alpha_kernel(wu_ref, wd_ref, au_ref, ad_ref):
    au_ref[...] = jnp.mean(jnp.abs(wu_ref[0])).reshape(1, 1, 1)
    ad_ref[...] = jnp.mean(jnp.abs(wd_ref[0])).reshape(1, 1, 1)


def _erf(x):
    a1, a2, a3, a4, a5 = (0.254829592, -0.284496736, 1.421413741,
                          -1.453152027, 1.061405429)
    p = 0.3275911
    ax = jnp.abs(x)
    t = 1.0 / (1.0 + p * ax)
    poly = t * (a1 + t * (a2 + t * (a3 + t * (a4 + t * a5))))
    y = 1.0 - poly * jnp.exp(-ax * ax)
    return jnp.sign(x) * y


def _gelu(x):
    return 0.5 * x * (1.0 + jax.lax.erf(x * (1.0 / math.sqrt(2.0))))


_NW = 32
_RPW = S // _NW


@functools.partial(
    pl.kernel,
    mesh=plsc.VectorSubcoreMesh(core_axis_name="c", subcore_axis_name="s"),
    out_type=[
        jax.ShapeDtypeStruct((S, D), jnp.float32),
        jax.ShapeDtypeStruct((S, D), jnp.float32),
    ],
    scratch_types=[
        pltpu.VMEM((_RPW,), jnp.int32),
        pltpu.VMEM((_RPW, D), jnp.float32),
        pltpu.SemaphoreType.DMA,
    ],
)
def _sc_combine(us_hbm, p0_hbm, p1_hbm, g0_hbm, g1_hbm, idx_v, rows_v, sem):
    wid = jax.lax.axis_index("s") * 2 + jax.lax.axis_index("c")
    base = wid * _RPW
    pltpu.sync_copy(p0_hbm.at[pl.ds(base, _RPW)], idx_v)
    pltpu.async_copy(us_hbm.at[idx_v], rows_v, sem).wait()
    pltpu.sync_copy(rows_v, g0_hbm.at[pl.ds(base, _RPW)])
    pltpu.sync_copy(p1_hbm.at[pl.ds(base, _RPW)], idx_v)
    pltpu.async_copy(us_hbm.at[idx_v], rows_v, sem).wait()
    pltpu.sync_copy(rows_v, g1_hbm.at[pl.ds(base, _RPW)])


TM = 256
P = 2 * S + E * TM
G = P // TM

TM = 256
P = 2 * S + E * TM
G = P // TM

_RPD = P // _NW
_RPD2 = _RPD // 2


@functools.partial(
    pl.kernel,
    mesh=plsc.VectorSubcoreMesh(core_axis_name="c", subcore_axis_name="s"),
    out_type=jax.ShapeDtypeStruct((P, D), jnp.float32),
    scratch_types=[
        pltpu.VMEM((_RPD2,), jnp.int32),
        pltpu.VMEM((_RPD2, D), jnp.float32),
        pltpu.SemaphoreType.DMA,
    ],
)
def _sc_dispatch(xn_hbm, tokp_hbm, xg_hbm, idx_v, rows_v, sem):
    wid = jax.lax.axis_index("s") * 2 + jax.lax.axis_index("c")
    for c in range(2):
        base = wid * _RPD + c * _RPD2
        pltpu.sync_copy(tokp_hbm.at[pl.ds(base, _RPD2)], idx_v)
        pltpu.async_copy(xn_hbm.at[idx_v], rows_v, sem).wait()
        pltpu.sync_copy(rows_v, xg_hbm.at[pl.ds(base, _RPD2)])


def _group_kernel(te_ref, xg_ref, wu_ref, wd_ref, au_ref, ad_ref,
                  w_ref, us_ref, wu_s, wd_s):
    i = pl.program_id(0)
    te_prev = te_ref[jnp.maximum(i - 1, 0)]
    te_cur = te_ref[i]

    @pl.when((i == 0) | (te_cur != te_prev))
    def _tern():
        au = au_ref[0, 0, 0]
        ad = ad_ref[0, 0, 0]

        def body(j, carry):
            ru = j * (DFF // 8)
            wu = wu_ref[0, pl.ds(ru, DFF // 8), :]
            wu_s[pl.ds(ru, DFF // 8), :] = jnp.where(
                jnp.abs(wu) > 0.5 * au, jnp.sign(wu), 0.0).astype(jnp.bfloat16)
            rd = j * (D // 8)
            wd = wd_ref[0, pl.ds(rd, D // 8), :]
            wd_s[pl.ds(rd, D // 8), :] = jnp.where(
                jnp.abs(wd) > 0.5 * ad, jnp.sign(wd), 0.0).astype(jnp.bfloat16)
            return carry

        jax.lax.fori_loop(0, 8, body, 0)

    xn = xg_ref[...]
    rms = jnp.sqrt(jnp.mean(xn * xn, axis=-1, keepdims=True))
    xs = (xn / (rms + 1e-8)).astype(jnp.bfloat16)
    u = jax.lax.dot_general(xs, wu_s[...], (((1,), (1,)), ((), ())),
                            preferred_element_type=jnp.float32)
    u = _gelu(u)
    urms = jnp.sqrt(jnp.mean(u * u, axis=-1, keepdims=True))
    u = (u / (urms + 1e-8)).astype(jnp.bfloat16)
    u = jax.lax.dot_general(u, wd_s[...], (((1,), (1,)), ((), ())),
                            preferred_element_type=jnp.float32)
    us_ref[...] = u * w_ref[...]


def kernel(x, in_proj_w, in_proj_b, out_w, out_b, ln1_w, ln1_b,
           ln2_w, ln2_b, W_up, W_down, router_w):
    residual = x
    h = _layernorm(x, ln1_w, ln1_b)
    h = _attention(h, in_proj_w, in_proj_b, out_w, out_b)
    x2 = h + residual
    x_norm = _layernorm(x2, ln2_w, ln2_b)
    h4 = x_norm @ router_w.T
    h4 = h4 / jnp.maximum(jnp.linalg.norm(h4, axis=-1, keepdims=True), 1e-12)
    dots = h4 @ _ROOTS.T
    bits = (dots >= 0).astype(jnp.int32)
    chamber = (bits[..., 0] + 2 * bits[..., 1]
               + 4 * bits[..., 2] + 8 * bits[..., 3])
    expert_indices = _C2E[chamber]
    confidence = jnp.min(jnp.abs(dots), axis=-1)
    w1 = 0.5 + 0.3 * jax.nn.sigmoid(confidence)
    e0 = expert_indices[..., 0].reshape(S, 1)
    e1 = expert_indices[..., 1].reshape(S, 1)
    w1 = w1.reshape(S, 1)

    alpha_up, alpha_dn = pl.pallas_call(
        _alpha_kernel,
        grid=(E,),
        in_specs=[
            pl.BlockSpec((1, DFF, D), lambda e: (e, 0, 0)),
            pl.BlockSpec((1, D, DFF), lambda e: (e, 0, 0)),
        ],
        out_specs=[
            pl.BlockSpec((1, 1, 1), lambda e: (e, 0, 0)),
            pl.BlockSpec((1, 1, 1), lambda e: (e, 0, 0)),
        ],
        out_shape=[
            jax.ShapeDtypeStruct((E, 1, 1), jnp.float32),
            jax.ShapeDtypeStruct((E, 1, 1), jnp.float32),
        ],
    )(W_up, W_down)

    e0f = e0.reshape(S)
    e1f = e1.reshape(S)
    w1f = w1.reshape(S)
    a = jnp.concatenate([e0f, e1f])
    onehot = (a[:, None] == jnp.arange(E, dtype=jnp.int32)[None, :]
              ).astype(jnp.int32)
    counts = jnp.sum(onehot, axis=0)
    rank = jnp.sum((jnp.cumsum(onehot, axis=0) - onehot) * onehot, axis=1)
    cap = ((counts + TM - 1) // TM) * TM
    padded_start = jnp.concatenate(
        [jnp.zeros((1,), jnp.int32), jnp.cumsum(cap)[:-1].astype(jnp.int32)])
    padded_pos = (jnp.sum(onehot * padded_start[None, :], axis=1) + rank
                  ).astype(jnp.int32)
    w_assign = jnp.concatenate([w1f, 1.0 - w1f])
    w_padded = jnp.zeros((P,), jnp.float32).at[padded_pos].set(
        w_assign).reshape(P, 1)
    p0 = padded_pos[:S]
    p1 = padded_pos[S:]
    tile_start = padded_start // TM
    tile_expert = (jnp.sum(
        jnp.arange(G, dtype=jnp.int32)[:, None] >= tile_start[None, :],
        axis=1) - 1).astype(jnp.int32)

    xn_flat = x_norm.reshape(S, D)
    xg = jnp.zeros((P, D), jnp.float32).at[padded_pos].set(
        jnp.concatenate([xn_flat, xn_flat], axis=0))

    us = pl.pallas_call(
        _group_kernel,
        grid_spec=pltpu.PrefetchScalarGridSpec(
            num_scalar_prefetch=1,
            grid=(G,),
            in_specs=[
                pl.BlockSpec((TM, D), lambda i, te: (i, 0)),
                pl.BlockSpec((1, DFF, D), lambda i, te: (te[i], 0, 0)),
                pl.BlockSpec((1, D, DFF), lambda i, te: (te[i], 0, 0)),
                pl.BlockSpec((1, 1, 1), lambda i, te: (te[i], 0, 0)),
                pl.BlockSpec((1, 1, 1), lambda i, te: (te[i], 0, 0)),
                pl.BlockSpec((TM, 1), lambda i, te: (i, 0)),
            ],
            out_specs=pl.BlockSpec((TM, D), lambda i, te: (i, 0)),
            scratch_shapes=[
                pltpu.VMEM((DFF, D), jnp.bfloat16),
                pltpu.VMEM((D, DFF), jnp.bfloat16),
            ],
        ),
        out_shape=jax.ShapeDtypeStruct((P, D), jnp.float32),
    )(tile_expert, xg, W_up, W_down, alpha_up, alpha_dn, w_padded)

    g0, g1 = _sc_combine(us, p0, p1)
    out = x2.reshape(S, D) + g0 + g1
    return out.reshape(B, S, D)


# speedup vs baseline: 1.2514x; 1.0227x over previous
import math

import jax
import jax.numpy as jnp
import numpy as np
from jax.experimental import pallas as pl
from jax.experimental.pallas import tpu as pltpu
from jax.experimental.pallas import tpu_sc as plsc
import functools

B, S, D = 1, 2048, 768
H = 12
DH = D // H
DFF = 2048
E = 8
PHI = (1.0 + math.sqrt(5.0)) / 2.0

_r = np.array(
    [[1.0, -1.0, 0.0, 0.0],
     [0.0, 1.0, -1.0, 0.0],
     [0.0, 0.0, 1.0, 0.0],
     [-0.5, 0.5, 0.5, PHI / 2.0]], dtype=np.float32)
_r = _r / np.linalg.norm(_r, axis=1, keepdims=True)
_ROOTS = jnp.asarray(_r)

_pairs = []
for _i in range(16):
    _e1 = _i % E
    _e2 = (_i // 2 + 1) % E
    if _e1 == _e2:
        _e2 = (_e2 + 1) % E
    _pairs.append([_e1, _e2])
_C2E = jnp.asarray(np.array(_pairs, dtype=np.int32))


def _layernorm(x, w, b):
    m = jnp.mean(x, axis=-1, keepdims=True)
    v = jnp.mean((x - m) ** 2, axis=-1, keepdims=True)
    return (x - m) / jnp.sqrt(v + 1e-5) * w + b


def _attention(x, in_proj_w, in_proj_b, out_w, out_b):
    qkv = x @ in_proj_w.T + in_proj_b
    q, k, v = jnp.split(qkv, 3, axis=-1)

    def sh(t):
        return t.reshape(B, S, H, DH).transpose(0, 2, 1, 3)

    q, k, v = sh(q), sh(k), sh(v)
    scores = (q @ k.transpose(0, 1, 3, 2)) / math.sqrt(DH)
    a = jax.nn.softmax(scores, axis=-1)
    o = a @ v
    o = o.transpose(0, 2, 1, 3).reshape(B, S, D)
    return o @ out_w.T + out_b


def _erf(x):
    a1, a2, a3, a4, a5 = (0.254829592, -0.284496736, 1.421413741,
                          -1.453152027, 1.061405429)
    p = 0.3275911
    ax = jnp.abs(x)
    t = 1.0 / (1.0 + p * ax)
    poly = t * (a1 + t * (a2 + t * (a3 + t * (a4 + t * a5))))
    y = 1.0 - poly * jnp.exp(-ax * ax)
    return jnp.sign(x) * y


def _gelu(x):
    return 0.5 * x * (1.0 + jax.lax.erf(x * (1.0 / math.sqrt(2.0))))


_NW = 32
_RPW = S // _NW


@functools.partial(
    pl.kernel,
    mesh=plsc.VectorSubcoreMesh(core_axis_name="c", subcore_axis_name="s"),
    out_type=[
        jax.ShapeDtypeStruct((S, D), jnp.float32),
        jax.ShapeDtypeStruct((S, D), jnp.float32),
    ],
    scratch_types=[
        pltpu.VMEM((_RPW,), jnp.int32),
        pltpu.VMEM((_RPW, D), jnp.float32),
        pltpu.SemaphoreType.DMA,
    ],
)
def _sc_combine(us_hbm, p0_hbm, p1_hbm, g0_hbm, g1_hbm, idx_v, rows_v, sem):
    wid = jax.lax.axis_index("s") * 2 + jax.lax.axis_index("c")
    base = wid * _RPW
    pltpu.sync_copy(p0_hbm.at[pl.ds(base, _RPW)], idx_v)
    pltpu.async_copy(us_hbm.at[idx_v], rows_v, sem).wait()
    pltpu.sync_copy(rows_v, g0_hbm.at[pl.ds(base, _RPW)])
    pltpu.sync_copy(p1_hbm.at[pl.ds(base, _RPW)], idx_v)
    pltpu.async_copy(us_hbm.at[idx_v], rows_v, sem).wait()
    pltpu.sync_copy(rows_v, g1_hbm.at[pl.ds(base, _RPW)])


TM = 256
P = 2 * S + E * TM
G = P // TM

TM = 256
P = 2 * S + E * TM
G = P // TM

_RPD = P // _NW
_RPD2 = _RPD // 2


@functools.partial(
    pl.kernel,
    mesh=plsc.VectorSubcoreMesh(core_axis_name="c", subcore_axis_name="s"),
    out_type=jax.ShapeDtypeStruct((P, D), jnp.float32),
    scratch_types=[
        pltpu.VMEM((_RPD2,), jnp.int32),
        pltpu.VMEM((_RPD2, D), jnp.float32),
        pltpu.SemaphoreType.DMA,
    ],
)
def _sc_dispatch(xn_hbm, tokp_hbm, xg_hbm, idx_v, rows_v, sem):
    wid = jax.lax.axis_index("s") * 2 + jax.lax.axis_index("c")
    for c in range(2):
        base = wid * _RPD + c * _RPD2
        pltpu.sync_copy(tokp_hbm.at[pl.ds(base, _RPD2)], idx_v)
        pltpu.async_copy(xn_hbm.at[idx_v], rows_v, sem).wait()
        pltpu.sync_copy(rows_v, xg_hbm.at[pl.ds(base, _RPD2)])


def _group_kernel(te_ref, xg_ref, wu_ref, wd_ref,
                  w_ref, us_ref, wu_s, wd_s):
    i = pl.program_id(0)
    te_prev = te_ref[jnp.maximum(i - 1, 0)]
    te_cur = te_ref[i]

    @pl.when((i == 0) | (te_cur != te_prev))
    def _tern():
        def sums(j, carry):
            su, sd = carry
            su = su + jnp.sum(jnp.abs(wu_ref[0, pl.ds(j * (DFF // 8), DFF // 8), :]))
            sd = sd + jnp.sum(jnp.abs(wd_ref[0, pl.ds(j * (D // 8), D // 8), :]))
            return su, sd

        su, sd = jax.lax.fori_loop(0, 8, sums, (0.0, 0.0))
        au = su / (DFF * D)
        ad = sd / (DFF * D)

        def body(j, carry):
            ru = j * (DFF // 8)
            wu = wu_ref[0, pl.ds(ru, DFF // 8), :]
            wu_s[pl.ds(ru, DFF // 8), :] = jnp.where(
                jnp.abs(wu) > 0.5 * au, jnp.sign(wu), 0.0).astype(jnp.bfloat16)
            rd = j * (D // 8)
            wd = wd_ref[0, pl.ds(rd, D // 8), :]
            wd_s[pl.ds(rd, D // 8), :] = jnp.where(
                jnp.abs(wd) > 0.5 * ad, jnp.sign(wd), 0.0).astype(jnp.bfloat16)
            return carry

        jax.lax.fori_loop(0, 8, body, 0)

    xn = xg_ref[...]
    rms = jnp.sqrt(jnp.mean(xn * xn, axis=-1, keepdims=True))
    xs = (xn / (rms + 1e-8)).astype(jnp.bfloat16)
    u = jax.lax.dot_general(xs, wu_s[...], (((1,), (1,)), ((), ())),
                            preferred_element_type=jnp.float32)
    u = _gelu(u)
    urms = jnp.sqrt(jnp.mean(u * u, axis=-1, keepdims=True))
    u = (u / (urms + 1e-8)).astype(jnp.bfloat16)
    u = jax.lax.dot_general(u, wd_s[...], (((1,), (1,)), ((), ())),
                            preferred_element_type=jnp.float32)
    us_ref[...] = u * w_ref[...]


def kernel(x, in_proj_w, in_proj_b, out_w, out_b, ln1_w, ln1_b,
           ln2_w, ln2_b, W_up, W_down, router_w):
    residual = x
    h = _layernorm(x, ln1_w, ln1_b)
    h = _attention(h, in_proj_w, in_proj_b, out_w, out_b)
    x2 = h + residual
    x_norm = _layernorm(x2, ln2_w, ln2_b)
    h4 = x_norm @ router_w.T
    h4 = h4 / jnp.maximum(jnp.linalg.norm(h4, axis=-1, keepdims=True), 1e-12)
    dots = h4 @ _ROOTS.T
    bits = (dots >= 0).astype(jnp.int32)
    chamber = (bits[..., 0] + 2 * bits[..., 1]
               + 4 * bits[..., 2] + 8 * bits[..., 3])
    expert_indices = _C2E[chamber]
    confidence = jnp.min(jnp.abs(dots), axis=-1)
    w1 = 0.5 + 0.3 * jax.nn.sigmoid(confidence)
    e0 = expert_indices[..., 0].reshape(S, 1)
    e1 = expert_indices[..., 1].reshape(S, 1)
    w1 = w1.reshape(S, 1)


    e0f = e0.reshape(S)
    e1f = e1.reshape(S)
    w1f = w1.reshape(S)
    a = jnp.concatenate([e0f, e1f])
    onehot = (a[:, None] == jnp.arange(E, dtype=jnp.int32)[None, :]
              ).astype(jnp.int32)
    counts = jnp.sum(onehot, axis=0)
    rank = jnp.sum((jnp.cumsum(onehot, axis=0) - onehot) * onehot, axis=1)
    cap = ((counts + TM - 1) // TM) * TM
    padded_start = jnp.concatenate(
        [jnp.zeros((1,), jnp.int32), jnp.cumsum(cap)[:-1].astype(jnp.int32)])
    padded_pos = (jnp.sum(onehot * padded_start[None, :], axis=1) + rank
                  ).astype(jnp.int32)
    w_assign = jnp.concatenate([w1f, 1.0 - w1f])
    w_padded = jnp.zeros((P,), jnp.float32).at[padded_pos].set(
        w_assign).reshape(P, 1)
    p0 = padded_pos[:S]
    p1 = padded_pos[S:]
    tile_start = padded_start // TM
    tile_expert = (jnp.sum(
        jnp.arange(G, dtype=jnp.int32)[:, None] >= tile_start[None, :],
        axis=1) - 1).astype(jnp.int32)

    xn_flat = x_norm.reshape(S, D)
    xg = jnp.zeros((P, D), jnp.float32).at[padded_pos].set(
        jnp.concatenate([xn_flat, xn_flat], axis=0))

    us = pl.pallas_call(
        _group_kernel,
        grid_spec=pltpu.PrefetchScalarGridSpec(
            num_scalar_prefetch=1,
            grid=(G,),
            in_specs=[
                pl.BlockSpec((TM, D), lambda i, te: (i, 0)),
                pl.BlockSpec((1, DFF, D), lambda i, te: (te[i], 0, 0)),
                pl.BlockSpec((1, D, DFF), lambda i, te: (te[i], 0, 0)),
                pl.BlockSpec((TM, 1), lambda i, te: (i, 0)),
            ],
            out_specs=pl.BlockSpec((TM, D), lambda i, te: (i, 0)),
            scratch_shapes=[
                pltpu.VMEM((DFF, D), jnp.bfloat16),
                pltpu.VMEM((D, DFF), jnp.bfloat16),
            ],
        ),
        out_shape=jax.ShapeDtypeStruct((P, D), jnp.float32),
    )(tile_expert, xg, W_up, W_down, w_padded)

    g0, g1 = _sc_combine(us, p0, p1)
    out = x2.reshape(S, D) + g0 + g1
    return out.reshape(B, S, D)
